# baseline (speedup 1.0000x reference)
import functools

import jax
import jax.numpy as jnp
from jax.experimental import pallas as pl
from jax.experimental.pallas import tpu as pltpu

N = 10000
D = 128
STRENGTH = 0.1
ALPHA = 0.15

BM = 200
BMC = 1000
NB = N // BM
NBC = N // BMC


def _body(feat0_ref, p0w1_ref, p0w2_ref, a0w1_ref, a0b1_ref, a0w2_ref,
          emb0_ref, emb1_ref, p1w1_ref, p1w2_ref, mp0_ref, mp1_ref, feat1_ref,
          out_ref, projbf_ref, prop0_ref, prop1_ref, sums_ref):
    i = pl.program_id(0)

    @pl.when(i == 0)
    def _init():
        low = jnp.dot(feat0_ref[...], p0w1_ref[...].T,
                      preferred_element_type=jnp.float32)
        proj = jnp.dot(low, p0w2_ref[...].T,
                       preferred_element_type=jnp.float32)
        projbf_ref[...] = proj.astype(jnp.bfloat16)

    @pl.when(i < NB)
    def _matmul():
        proj = projbf_ref[...]
        p0 = jnp.dot(mp0_ref[...].astype(jnp.bfloat16), proj,
                     preferred_element_type=jnp.float32)
        p1 = jnp.dot(mp1_ref[...].astype(jnp.bfloat16), proj,
                     preferred_element_type=jnp.float32)
        rows = pl.ds(i * BM, BM)
        prop0_ref[rows, :] = p0
        prop1_ref[rows, :] = p1

    @pl.when(i == NB)
    def _attention():
        s0 = jnp.sum(prop0_ref[...], axis=0, keepdims=True)
        s1 = jnp.sum(prop1_ref[...], axis=0, keepdims=True)
        means = jnp.concatenate([s0, s1], axis=0) * (1.0 / N)
        h = jnp.tanh(jnp.dot(means, a0w1_ref[...].T,
                             preferred_element_type=jnp.float32)
                     + a0b1_ref[...])
        logits = jnp.dot(h, a0w2_ref[...].T,
                         preferred_element_type=jnp.float32)
        m = jnp.max(logits, axis=0, keepdims=True)
        e = jnp.exp(logits - m)
        w = e / jnp.sum(e, axis=0, keepdims=True)
        sums_ref[0:2, 0:1] = w

    @pl.when(i >= NB)
    def _combine():
        j = i - NB
        rows = pl.ds(j * BMC, BMC)
        propagated = (prop0_ref[rows, :] * sums_ref[0:1, 0:1]
                      + prop1_ref[rows, :] * sums_ref[1:2, 0:1])
        gate0 = jax.nn.sigmoid(emb0_ref[...])
        meta_signal = STRENGTH * (propagated * gate0)
        out_ref[0] = ((1.0 + ALPHA) * feat0_ref[rows, :]
                      + (1.0 - ALPHA) * meta_signal)

        f1 = feat1_ref[...]
        t1 = jnp.dot(jnp.dot(f1, p1w1_ref[...].T,
                             preferred_element_type=jnp.float32),
                     p1w2_ref[...].T, preferred_element_type=jnp.float32)
        gate1 = jax.nn.sigmoid(emb1_ref[...])
        out_ref[1] = f1 + STRENGTH * (t1 * gate1)


@functools.partial(jax.jit, static_argnames=("interpret",))
def _run(feat0, feat1, mp0, mp1, emb0, emb1, p0w1, p0w2, p1w1, p1w2,
         a0w1, a0b1, a0w2, interpret=False):
    whole = lambda shape: pl.BlockSpec(shape, lambda i: (0,) * len(shape))
    a0b1_2d = a0b1.reshape(1, -1)

    def mp_idx(i):
        return (jnp.minimum(i, NB - 1), 0)

    def f1_idx(i):
        return (jnp.maximum(i - NB, 0), 0)

    def out_idx(i):
        return (0, jnp.maximum(i - NB, 0), 0)

    out = pl.pallas_call(
        _body,
        grid=(NB + NBC,),
        in_specs=[
            whole((N, D)),
            whole(p0w1.shape),
            whole(p0w2.shape),
            whole(a0w1.shape),
            whole((1, a0b1.shape[0])),
            whole(a0w2.shape),
            whole(emb0.shape),
            whole(emb1.shape),
            whole(p1w1.shape),
            whole(p1w2.shape),
            pl.BlockSpec((BM, N), mp_idx),
            pl.BlockSpec((BM, N), mp_idx),
            pl.BlockSpec((BMC, D), f1_idx),
        ],
        out_specs=pl.BlockSpec((2, BMC, D), out_idx),
        out_shape=jax.ShapeDtypeStruct((2, N, D), jnp.float32),
        scratch_shapes=[
            pltpu.VMEM((N, D), jnp.bfloat16),
            pltpu.VMEM((N, D), jnp.float32),
            pltpu.VMEM((N, D), jnp.float32),
            pltpu.VMEM((8, D), jnp.float32),
        ],
        compiler_params=pltpu.CompilerParams(
            dimension_semantics=("arbitrary",),
            vmem_limit_bytes=112 * 1024 * 1024,
        ),
        interpret=interpret,
    )(feat0, p0w1, p0w2, a0w1, a0b1_2d, a0w2, emb0, emb1, p1w1, p1w2,
      mp0, mp1, feat1)
    return out


def kernel(feat0, feat1, mp0, mp1, emb0, emb1, p0w1, p0w2, p1w1, p1w2,
           a0w1, a0b1, a0w2):
    return _run(feat0, feat1, mp0, mp1, emb0, emb1, p0w1, p0w2, p1w1, p1w2,
                a0w1, a0b1, a0w2)


# speedup vs baseline: 1.0088x; 1.0088x over previous
import functools

import jax
import jax.numpy as jnp
from jax.experimental import pallas as pl
from jax.experimental.pallas import tpu as pltpu

N = 10000
D = 128
STRENGTH = 0.1
ALPHA = 0.15

BM = 200
BMC = 1000
NB = N // BM
NBC = N // BMC


def _body(feat0_ref, p0w1_ref, p0w2_ref, a0w1_ref, a0b1_ref, a0w2_ref,
          emb0_ref, emb1_ref, p1w1_ref, p1w2_ref, mp0_ref, mp1_ref, feat1_ref,
          out_ref, projbf_ref, prop0_ref, prop1_ref, sums_ref):
    i = pl.program_id(0)

    @pl.when(i == 0)
    def _init():
        w128 = jnp.dot(p0w1_ref[...].T, p0w2_ref[...].T,
                       preferred_element_type=jnp.float32)
        projbf_ref[...] = jnp.dot(
            feat0_ref[...].astype(jnp.bfloat16), w128.astype(jnp.bfloat16),
            preferred_element_type=jnp.float32).astype(jnp.bfloat16)
        sums_ref[...] = jnp.zeros_like(sums_ref)

    @pl.when(i < NB)
    def _matmul():
        proj = projbf_ref[...]
        p0 = jnp.dot(mp0_ref[...].astype(jnp.bfloat16), proj,
                     preferred_element_type=jnp.float32)
        p1 = jnp.dot(mp1_ref[...].astype(jnp.bfloat16), proj,
                     preferred_element_type=jnp.float32)
        rows = pl.ds(i * BM, BM)
        prop0_ref[rows, :] = p0
        prop1_ref[rows, :] = p1
        sums_ref[0:1, :] += jnp.sum(p0, axis=0, keepdims=True)
        sums_ref[1:2, :] += jnp.sum(p1, axis=0, keepdims=True)

    @pl.when(i < NBC)
    def _out1():
        w128 = jnp.dot(p1w1_ref[...].T, p1w2_ref[...].T,
                       preferred_element_type=jnp.float32)
        f1 = feat1_ref[...]
        t1 = jnp.dot(f1, w128, preferred_element_type=jnp.float32)
        gate1 = jax.nn.sigmoid(emb1_ref[...])
        out_ref[0] = f1 + STRENGTH * (t1 * gate1)

    @pl.when(i >= NB)
    def _combine():
        j = i - NB
        means = sums_ref[0:2, :] * (1.0 / N)
        h = jnp.tanh(jnp.dot(means, a0w1_ref[...].T,
                             preferred_element_type=jnp.float32)
                     + a0b1_ref[...])
        logits = jnp.dot(h, a0w2_ref[...].T,
                         preferred_element_type=jnp.float32)
        m = jnp.max(logits, axis=0, keepdims=True)
        e = jnp.exp(logits - m)
        w = e / jnp.sum(e, axis=0, keepdims=True)

        rows = pl.ds(j * BMC, BMC)
        propagated = (prop0_ref[rows, :] * w[0:1, 0:1]
                      + prop1_ref[rows, :] * w[1:2, 0:1])
        gate0 = jax.nn.sigmoid(emb0_ref[...])
        meta_signal = STRENGTH * (propagated * gate0)
        out_ref[0] = ((1.0 + ALPHA) * feat0_ref[rows, :]
                      + (1.0 - ALPHA) * meta_signal)


@functools.partial(jax.jit, static_argnames=("interpret",))
def _run(feat0, feat1, mp0, mp1, emb0, emb1, p0w1, p0w2, p1w1, p1w2,
         a0w1, a0b1, a0w2, interpret=False):
    whole = lambda shape: pl.BlockSpec(shape, lambda i: (0,) * len(shape))
    a0b1_2d = a0b1.reshape(1, -1)

    def mp_idx(i):
        return (jnp.minimum(i, NB - 1), 0)

    def f1_idx(i):
        return (jnp.minimum(i, NBC - 1), 0)

    def out_idx(i):
        return (jnp.where(i < NB, 1, 0),
                jnp.where(i < NB, jnp.minimum(i, NBC - 1), i - NB),
                0)

    out = pl.pallas_call(
        _body,
        grid=(NB + NBC,),
        in_specs=[
            whole((N, D)),
            whole(p0w1.shape),
            whole(p0w2.shape),
            whole(a0w1.shape),
            whole((1, a0b1.shape[0])),
            whole(a0w2.shape),
            whole(emb0.shape),
            whole(emb1.shape),
            whole(p1w1.shape),
            whole(p1w2.shape),
            pl.BlockSpec((BM, N), mp_idx),
            pl.BlockSpec((BM, N), mp_idx),
            pl.BlockSpec((BMC, D), f1_idx),
        ],
        out_specs=pl.BlockSpec((1, BMC, D), out_idx),
        out_shape=jax.ShapeDtypeStruct((2, N, D), jnp.float32),
        scratch_shapes=[
            pltpu.VMEM((N, D), jnp.bfloat16),
            pltpu.VMEM((N, D), jnp.float32),
            pltpu.VMEM((N, D), jnp.float32),
            pltpu.VMEM((8, D), jnp.float32),
        ],
        compiler_params=pltpu.CompilerParams(
            dimension_semantics=("arbitrary",),
        ),
        interpret=interpret,
    )(feat0, p0w1, p0w2, a0w1, a0b1_2d, a0w2, emb0, emb1, p1w1, p1w2,
      mp0, mp1, feat1)
    return out


def kernel(feat0, feat1, mp0, mp1, emb0, emb1, p0w1, p0w2, p1w1, p1w2,
           a0w1, a0b1, a0w2):
    return _run(feat0, feat1, mp0, mp1, emb0, emb1, p0w1, p0w2, p1w1, p1w2,
                a0w1, a0b1, a0w2)


# speedup vs baseline: 1.0254x; 1.0165x over previous
import functools

import jax
import jax.numpy as jnp
from jax.experimental import pallas as pl
from jax.experimental.pallas import tpu as pltpu

N = 10000
D = 128
STRENGTH = 0.1
ALPHA = 0.15

BM = 200
BMC = 2000
NB = N // BM
NBC = N // BMC


def _body(feat0_ref, p0w1_ref, p0w2_ref, a0w1_ref, a0b1_ref, a0w2_ref,
          emb0_ref, emb1_ref, p1w1_ref, p1w2_ref, mp0_ref, mp1_ref, feat1_ref,
          out_ref, proj_ref, prop0_ref, prop1_ref, sums_ref):
    i = pl.program_id(0)

    @pl.when(i == 0)
    def _init():
        w128 = jnp.dot(p0w1_ref[...].T, p0w2_ref[...].T,
                       preferred_element_type=jnp.float32)
        proj_ref[...] = jnp.dot(feat0_ref[...], w128,
                                preferred_element_type=jnp.float32)
        sums_ref[...] = jnp.zeros_like(sums_ref)

    @pl.when(i < NB)
    def _matmul():
        proj = proj_ref[...]
        p0 = jnp.dot(mp0_ref[...], proj, preferred_element_type=jnp.float32)
        p1 = jnp.dot(mp1_ref[...], proj, preferred_element_type=jnp.float32)
        rows = pl.ds(i * BM, BM)
        prop0_ref[rows, :] = p0.astype(jnp.bfloat16)
        prop1_ref[rows, :] = p1.astype(jnp.bfloat16)
        sums_ref[0:1, :] += jnp.sum(p0, axis=0, keepdims=True)
        sums_ref[1:2, :] += jnp.sum(p1, axis=0, keepdims=True)

    @pl.when(i >= NB)
    def _combine():
        j = i - NB
        means = sums_ref[0:2, :] * (1.0 / N)
        h = jnp.tanh(jnp.dot(means, a0w1_ref[...].T,
                             preferred_element_type=jnp.float32)
                     + a0b1_ref[...])
        logits = jnp.dot(h, a0w2_ref[...].T,
                         preferred_element_type=jnp.float32)
        m = jnp.max(logits, axis=0, keepdims=True)
        e = jnp.exp(logits - m)
        w = e / jnp.sum(e, axis=0, keepdims=True)

        rows = pl.ds(j * BMC, BMC)
        propagated = (prop0_ref[rows, :].astype(jnp.float32) * w[0:1, 0:1]
                      + prop1_ref[rows, :].astype(jnp.float32)
                      * w[1:2, 0:1])
        gate0 = jax.nn.sigmoid(emb0_ref[...])
        meta_signal = STRENGTH * (propagated * gate0)
        out_ref[0] = ((1.0 + ALPHA) * feat0_ref[rows, :]
                      + (1.0 - ALPHA) * meta_signal)

        w1c = jnp.dot(p1w1_ref[...].T, p1w2_ref[...].T,
                      preferred_element_type=jnp.float32)
        f1 = feat1_ref[...]
        t1 = jnp.dot(f1, w1c, preferred_element_type=jnp.float32)
        gate1 = jax.nn.sigmoid(emb1_ref[...])
        out_ref[1] = f1 + STRENGTH * (t1 * gate1)


@functools.partial(jax.jit, static_argnames=("interpret",))
def _run(feat0, feat1, mp0, mp1, emb0, emb1, p0w1, p0w2, p1w1, p1w2,
         a0w1, a0b1, a0w2, interpret=False):
    whole = lambda shape: pl.BlockSpec(shape, lambda i: (0,) * len(shape))
    a0b1_2d = a0b1.reshape(1, -1)

    def mp_idx(i):
        return (jnp.minimum(i, NB - 1), 0)

    def f1_idx(i):
        return (jnp.maximum(i - NB, 0), 0)

    def out_idx(i):
        return (0, jnp.maximum(i - NB, 0), 0)

    out = pl.pallas_call(
        _body,
        grid=(NB + NBC,),
        in_specs=[
            whole((N, D)),
            whole(p0w1.shape),
            whole(p0w2.shape),
            whole(a0w1.shape),
            whole((1, a0b1.shape[0])),
            whole(a0w2.shape),
            whole(emb0.shape),
            whole(emb1.shape),
            whole(p1w1.shape),
            whole(p1w2.shape),
            pl.BlockSpec((BM, N), mp_idx),
            pl.BlockSpec((BM, N), mp_idx),
            pl.BlockSpec((BMC, D), f1_idx),
        ],
        out_specs=pl.BlockSpec((2, BMC, D), out_idx),
        out_shape=jax.ShapeDtypeStruct((2, N, D), jnp.float32),
        scratch_shapes=[
            pltpu.VMEM((N, D), jnp.float32),
            pltpu.VMEM((N, D), jnp.bfloat16),
            pltpu.VMEM((N, D), jnp.bfloat16),
            pltpu.VMEM((8, D), jnp.float32),
        ],
        compiler_params=pltpu.CompilerParams(
            dimension_semantics=("arbitrary",),
        ),
        interpret=interpret,
    )(feat0, p0w1, p0w2, a0w1, a0b1_2d, a0w2, emb0, emb1, p1w1, p1w2,
      mp0, mp1, feat1)
    return out


def kernel(feat0, feat1, mp0, mp1, emb0, emb1, p0w1, p0w2, p1w1, p1w2,
           a0w1, a0b1, a0w2):
    return _run(feat0, feat1, mp0, mp1, emb0, emb1, p0w1, p0w2, p1w1, p1w2,
                a0w1, a0b1, a0w2)


# speedup vs baseline: 1.0259x; 1.0005x over previous
import functools

import jax
import jax.numpy as jnp
from jax.experimental import pallas as pl
from jax.experimental.pallas import tpu as pltpu

N = 10000
D = 128
STRENGTH = 0.1
ALPHA = 0.15

BM = 200
BMC = 2000
NB = N // BM
NBC = N // BMC


def _body(feat0_ref, p0w1_ref, p0w2_ref, a0w1_ref, a0b1_ref, a0w2_ref,
          emb0_ref, emb1_ref, p1w1_ref, p1w2_ref, mp0_ref, mp1_ref, feat1_ref,
          out_ref, proj_ref, prop0_ref, prop1_ref, sums_ref):
    i = pl.program_id(0)

    @pl.when(i == 0)
    def _init():
        w128 = jnp.dot(p0w1_ref[...].T, p0w2_ref[...].T,
                       preferred_element_type=jnp.float32)
        proj_ref[...] = jnp.dot(feat0_ref[...], w128,
                                preferred_element_type=jnp.float32)
        sums_ref[...] = jnp.zeros_like(sums_ref)

    @pl.when(i < NB)
    def _matmul():
        proj = proj_ref[...]
        p0 = jnp.dot(mp0_ref[...], proj, preferred_element_type=jnp.float32)
        p1 = jnp.dot(mp1_ref[...], proj, preferred_element_type=jnp.float32)
        rows = pl.ds(i * BM, BM)
        prop0_ref[rows, :] = p0
        prop1_ref[rows, :] = p1
        sums_ref[0:1, :] += jnp.sum(p0, axis=0, keepdims=True)
        sums_ref[1:2, :] += jnp.sum(p1, axis=0, keepdims=True)

    @pl.when(i >= NB)
    def _combine():
        j = i - NB
        means = sums_ref[0:2, :] * (1.0 / N)
        h = jnp.tanh(jnp.dot(means, a0w1_ref[...].T,
                             preferred_element_type=jnp.float32)
                     + a0b1_ref[...])
        logits = jnp.dot(h, a0w2_ref[...].T,
                         preferred_element_type=jnp.float32)
        m = jnp.max(logits, axis=0, keepdims=True)
        e = jnp.exp(logits - m)
        w = e / jnp.sum(e, axis=0, keepdims=True)

        rows = pl.ds(j * BMC, BMC)
        propagated = (prop0_ref[rows, :] * w[0:1, 0:1]
                      + prop1_ref[rows, :] * w[1:2, 0:1])
        gate0 = jax.nn.sigmoid(emb0_ref[...])
        meta_signal = STRENGTH * (propagated * gate0)
        out_ref[0] = ((1.0 + ALPHA) * feat0_ref[rows, :]
                      + (1.0 - ALPHA) * meta_signal)

        w1c = jnp.dot(p1w1_ref[...].T, p1w2_ref[...].T,
                      preferred_element_type=jnp.float32)
        f1 = feat1_ref[...]
        t1 = jnp.dot(f1, w1c, preferred_element_type=jnp.float32)
        gate1 = jax.nn.sigmoid(emb1_ref[...])
        out_ref[1] = f1 + STRENGTH * (t1 * gate1)


@functools.partial(jax.jit, static_argnames=("interpret",))
def _run(feat0, feat1, mp0, mp1, emb0, emb1, p0w1, p0w2, p1w1, p1w2,
         a0w1, a0b1, a0w2, interpret=False):
    whole = lambda shape: pl.BlockSpec(shape, lambda i: (0,) * len(shape))
    a0b1_2d = a0b1.reshape(1, -1)

    def mp_idx(i):
        return (jnp.minimum(i, NB - 1), 0)

    def f1_idx(i):
        return (jnp.maximum(i - NB, 0), 0)

    def out_idx(i):
        return (0, jnp.maximum(i - NB, 0), 0)

    out = pl.pallas_call(
        _body,
        grid=(NB + NBC,),
        in_specs=[
            whole((N, D)),
            whole(p0w1.shape),
            whole(p0w2.shape),
            whole(a0w1.shape),
            whole((1, a0b1.shape[0])),
            whole(a0w2.shape),
            whole(emb0.shape),
            whole(emb1.shape),
            whole(p1w1.shape),
            whole(p1w2.shape),
            pl.BlockSpec((BM, N), mp_idx),
            pl.BlockSpec((BM, N), mp_idx),
            pl.BlockSpec((BMC, D), f1_idx),
        ],
        out_specs=pl.BlockSpec((2, BMC, D), out_idx),
        out_shape=jax.ShapeDtypeStruct((2, N, D), jnp.float32),
        scratch_shapes=[
            pltpu.VMEM((N, D), jnp.float32),
            pltpu.VMEM((N, D), jnp.float32),
            pltpu.VMEM((N, D), jnp.float32),
            pltpu.VMEM((8, D), jnp.float32),
        ],
        compiler_params=pltpu.CompilerParams(
            dimension_semantics=("arbitrary",),
        ),
        interpret=interpret,
    )(feat0, p0w1, p0w2, a0w1, a0b1_2d, a0w2, emb0, emb1, p1w1, p1w2,
      mp0, mp1, feat1)
    return out


def kernel(feat0, feat1, mp0, mp1, emb0, emb1, p0w1, p0w2, p1w1, p1w2,
           a0w1, a0b1, a0w2):
    return _run(feat0, feat1, mp0, mp1, emb0, emb1, p0w1, p0w2, p1w1, p1w2,
                a0w1, a0b1, a0w2)
